# baseline (speedup 1.0000x reference)
import jax
import jax.numpy as jnp
from jax.experimental import pallas as pl
from jax.experimental.pallas import tpu as pltpu

_P = 16
_K = 64
_N = 2048
_D = 512
_PD = _P * _D
_BLK = 512
_CBLK = 256
_NSTRIP = _N // _BLK
_NJ = _N // _CBLK
_ITERS_PER_STEP = 4

_HI_KEY = 0x3FC00000
_LO_KEY = -0x3FC00001 - 1


def _body(ctx_ref, w_ref, out_ref, vr_ref, vc_ref, inv_ref, mir_ref,
          acc_ref, lo_ref, hi_ref):
    i = pl.program_id(0)
    j = pl.program_id(1)
    w = w_ref[...]

    @pl.when(jnp.logical_and(i == 0, j == 0))
    def _norms():
        c = ctx_ref[...]
        n2 = jax.lax.dot_general(
            c * c, w * w, (((1,), (1,)), ((), ())),
            preferred_element_type=jnp.float32,
            precision=jax.lax.Precision.HIGHEST)
        inv_ref[...] = 1.0 / jnp.maximum(jnp.sqrt(n2), 1e-12)

    @pl.when(jnp.logical_and(i < _NSTRIP, j == 0))
    def _build_rows():
        c = ctx_ref[pl.ds(i * _BLK, _BLK), :]
        inv = inv_ref[pl.ds(i * _BLK, _BLK), :]
        for p in range(_P):
            vr_ref[:, p * _D:(p + 1) * _D] = (
                c * w[p][None, :] * inv[:, p][:, None])

    par = jax.lax.rem(i, 2)

    @pl.when(jnp.logical_and(i < _NSTRIP, j >= 2 * i))
    def _upper():
        cj = ctx_ref[pl.ds(j * _CBLK, _CBLK), :]
        invj = inv_ref[pl.ds(j * _CBLK, _CBLK), :]
        for p in range(_P):
            vc_ref[:, p * _D:(p + 1) * _D] = (
                cj * w[p][None, :] * invj[:, p][:, None])

        part = jax.lax.dot_general(
            vr_ref[...], vc_ref[...], (((1,), (1,)), ((), ())),
            preferred_element_type=jnp.float32,
            precision=jax.lax.Precision.DEFAULT) * (1.0 / _P)
        acc_ref[par, :, pl.ds(j * _CBLK, _CBLK)] = part

        @pl.when(j >= 2 * i + 2)
        def _stash_mirror():
            mir_ref[pl.ds(j * _CBLK - _BLK, _CBLK), pl.ds(i * _BLK, _BLK)] = (
                jnp.transpose(part))

    @pl.when(jnp.logical_and(i < _NSTRIP, j < 2 * i))
    def _copy_mirror():
        acc_ref[par, :, pl.ds(j * _CBLK, _CBLK)] = (
            mir_ref[pl.ds(i * _BLK - _BLK, _BLK), pl.ds(j * _CBLK, _CBLK)])

    @pl.when(i > 0)
    def _search():
        att = acc_ref[1 - par]

        @pl.when(j == 0)
        def _init_bounds():
            lo_ref[...] = jnp.full((_BLK, 1), _LO_KEY, jnp.int32)
            hi_ref[...] = jnp.full((_BLK, 1), _HI_KEY, jnp.int32)

        def unmap(m):
            b = jnp.where(m >= 0, m, m ^ 0x7FFFFFFF)
            return jax.lax.bitcast_convert_type(b, jnp.float32)

        def step(_, lh):
            lo, hi = lh
            mid = (lo + hi) >> 1
            cnt = jnp.sum((att > unmap(mid)).astype(jnp.float32), axis=1,
                          keepdims=True)
            ge = cnt >= float(_K)
            return jnp.where(ge, mid, lo), jnp.where(ge, hi, mid)

        lo, hi = jax.lax.fori_loop(0, _ITERS_PER_STEP, step,
                                   (lo_ref[...], hi_ref[...]))
        lo_ref[...] = lo
        hi_ref[...] = hi

        @pl.when(j == _NJ - 1)
        def _mask_and_write():
            out_ref[...] = jnp.where(att > unmap(lo), att, 0.0)


@jax.jit
def kernel(context, W):
    return pl.pallas_call(
        _body,
        grid=(_NSTRIP + 1, _NJ),
        in_specs=[
            pl.BlockSpec((_N, _D), lambda i, j: (0, 0)),
            pl.BlockSpec((_P, _D), lambda i, j: (0, 0)),
        ],
        out_specs=pl.BlockSpec((_BLK, _N),
                               lambda i, j: (jnp.maximum(i, 1) - 1, 0)),
        out_shape=jax.ShapeDtypeStruct((_N, _N), jnp.float32),
        scratch_shapes=[
            pltpu.VMEM((_BLK, _PD), jnp.float32),
            pltpu.VMEM((_CBLK, _PD), jnp.float32),
            pltpu.VMEM((_N, _P), jnp.float32),
            pltpu.VMEM((_N - _BLK, _N - _BLK), jnp.float32),
            pltpu.VMEM((2, _BLK, _N), jnp.float32),
            pltpu.VMEM((_BLK, 1), jnp.int32),
            pltpu.VMEM((_BLK, 1), jnp.int32),
        ],
    )(context, W)
